# baseline (speedup 1.0000x reference)
import functools

import jax
import jax.numpy as jnp
from jax import lax
from jax.experimental import pallas as pl
from jax.experimental.pallas import tpu as pltpu
from jax.experimental.pallas import tpu_sc as plsc

N_NODES = 10000
N_EDGES = 320000
D = 128

NC = 2
NS = 16
NW = NC * NS
WIN = 128
NWTOT = N_EDGES // WIN
WPS = NWTOT // NW
NXTRA = NWTOT - WPS * NW
NTRI = WPS // 3

NPS_A = 632
NPS_B = N_NODES - (NS - 1) * NPS_A


def _sc_gather_segment_sum(x, ref_a, backref):
    mesh = plsc.VectorSubcoreMesh(core_axis_name="c", subcore_axis_name="s")

    @functools.partial(
        pl.kernel,
        out_type=jax.ShapeDtypeStruct((NC, N_NODES, D), jnp.float32),
        mesh=mesh,
        scratch_types=[
            pltpu.VMEM_SHARED((N_NODES, D), jnp.float32),
        ]
        + [pltpu.VMEM((WIN,), jnp.int32)] * 6
        + [pltpu.VMEM((WIN, D), jnp.float32)] * 3
        + [pltpu.SemaphoreType.DMA] * 7,
    )
    def k(x_hbm, ra_hbm, br_hbm, out_hbm, acc,
          ia0, ib0, ia1, ib1, ia2, ib2, r0, r1, r2,
          g0, g1, g2, t0, t1, t2, gi):
        c = lax.axis_index("c")
        s = lax.axis_index("s")
        wid = c * NS + s

        @pl.loop(0, WIN)
        def _(i):
            @pl.loop(0, D, step=16)
            def _(j):
                r0[i, pl.ds(j, 16)] = jnp.zeros((16,), jnp.float32)

        nbase = pl.multiple_of(s * NPS_A, 8)

        def zero_rows(base, nrows):
            @pl.loop(0, nrows // WIN)
            def _(t):
                pltpu.sync_copy(r0, acc.at[pl.ds(base + t * WIN, WIN)])
            rem = nrows - (nrows // WIN) * WIN
            if rem:
                pltpu.sync_copy(r0.at[pl.ds(0, rem)],
                                acc.at[pl.ds(base + (nrows // WIN) * WIN, rem)])

        @pl.when(s < NS - 1)
        def _():
            zero_rows(nbase, NPS_A)

        @pl.when(s == NS - 1)
        def _():
            zero_rows(nbase, NPS_B)

        plsc.subcore_barrier()

        ebase = wid * (WPS * WIN)

        @pl.loop(0, NTRI)
        def _(p):
            off = ebase + p * (3 * WIN)
            hs = []
            for (ia, ib, d) in ((ia0, ib0, 0), (ia1, ib1, 1), (ia2, ib2, 2)):
                hs.append(pltpu.async_copy(
                    ra_hbm.at[pl.ds(off + d * WIN, WIN)], ia, gi))
                hs.append(pltpu.async_copy(
                    br_hbm.at[pl.ds(off + d * WIN, WIN)], ib, gi))
            for h in hs:
                h.wait()
            cp0 = pltpu.async_copy(x_hbm.at[ia0], r0, g0)
            cp1 = pltpu.async_copy(x_hbm.at[ia1], r1, g1)
            cp2 = pltpu.async_copy(x_hbm.at[ia2], r2, g2)
            cp0.wait()
            pltpu.sync_copy(r0, acc.at[ib0], add=True)
            cp1.wait()
            pltpu.sync_copy(r1, acc.at[ib1], add=True)
            cp2.wait()
            pltpu.sync_copy(r2, acc.at[ib2], add=True)

        @pl.when(wid >= NW - NXTRA)
        def _():
            off = (WPS * NW + (wid - (NW - NXTRA))) * WIN
            pltpu.sync_copy(ra_hbm.at[pl.ds(off, WIN)], ia0)
            pltpu.sync_copy(br_hbm.at[pl.ds(off, WIN)], ib0)
            pltpu.async_copy(x_hbm.at[ia0], r0, g0).wait()
            pltpu.sync_copy(r0, acc.at[ib0], add=True)

        plsc.subcore_barrier()

        @pl.when(s < NS - 1)
        def _():
            pltpu.sync_copy(acc.at[pl.ds(nbase, NPS_A)],
                            out_hbm.at[c, pl.ds(nbase, NPS_A)])

        @pl.when(s == NS - 1)
        def _():
            pltpu.sync_copy(acc.at[pl.ds(nbase, NPS_B)],
                            out_hbm.at[c, pl.ds(nbase, NPS_B)])

    return k(x, ref_a, backref)


def _tc_combine(x, s0, s1, w, w_prop, b):
    br = 1000

    def body(x_ref, s0_ref, s1_ref, w_ref, wp_ref, b_ref, o_ref):
        acc = jnp.dot(x_ref[...], w_ref[...], preferred_element_type=jnp.float32)
        conv = s0_ref[...] + s1_ref[...]
        acc += jnp.dot(conv, wp_ref[...], preferred_element_type=jnp.float32)
        o_ref[...] = jnp.maximum(acc + b_ref[...], 0.0)

    return pl.pallas_call(
        body,
        grid=(N_NODES // br,),
        in_specs=[
            pl.BlockSpec((br, D), lambda i: (i, 0)),
            pl.BlockSpec((br, D), lambda i: (i, 0)),
            pl.BlockSpec((br, D), lambda i: (i, 0)),
            pl.BlockSpec((D, D), lambda i: (0, 0)),
            pl.BlockSpec((D, D), lambda i: (0, 0)),
            pl.BlockSpec((1, D), lambda i: (0, 0)),
        ],
        out_specs=pl.BlockSpec((br, D), lambda i: (i, 0)),
        out_shape=jax.ShapeDtypeStruct((N_NODES, D), jnp.float32),
    )(x, s0, s1, w, w_prop, b.reshape(1, D))


def kernel(X, ref_a, backref, e_map, v_count, W, W_prop, b):
    partials = _sc_gather_segment_sum(X, ref_a, backref)
    X_out = _tc_combine(X, partials[0], partials[1], W, W_prop, b)
    return (X_out, ref_a, backref, e_map, v_count)


# speedup vs baseline: 1.0629x; 1.0629x over previous
import functools

import jax
import jax.numpy as jnp
from jax import lax
from jax.experimental import pallas as pl
from jax.experimental.pallas import tpu as pltpu
from jax.experimental.pallas import tpu_sc as plsc

N_NODES = 10000
N_EDGES = 320000
D = 128

NC = 2
NS = 16
NW = NC * NS
WIN = 128
NWTOT = N_EDGES // WIN
WPS = NWTOT // NW
NXTRA = NWTOT - WPS * NW
NTRI = WPS // 3
NDUO = NTRI // 2

NPS_A = 632
NPS_B = N_NODES - (NS - 1) * NPS_A


def _sc_gather_segment_sum(x, ref_a, backref):
    mesh = plsc.VectorSubcoreMesh(core_axis_name="c", subcore_axis_name="s")

    @functools.partial(
        pl.kernel,
        out_type=jax.ShapeDtypeStruct((NC, N_NODES, D), jnp.float32),
        mesh=mesh,
        scratch_types=[
            pltpu.VMEM_SHARED((N_NODES, D), jnp.float32),
        ]
        + [pltpu.VMEM((WIN,), jnp.int32)] * 12
        + [pltpu.VMEM((WIN, D), jnp.float32)] * 3
        + [pltpu.SemaphoreType.DMA] * 5,
    )
    def k(x_hbm, ra_hbm, br_hbm, out_hbm, acc,
          iaA0, ibA0, iaA1, ibA1, iaA2, ibA2,
          iaB0, ibB0, iaB1, ibB1, iaB2, ibB2,
          r0, r1, r2, g0, g1, g2, giA, giB):
        c = lax.axis_index("c")
        s = lax.axis_index("s")
        wid = c * NS + s

        @pl.loop(0, WIN)
        def _(i):
            @pl.loop(0, D, step=16)
            def _(j):
                r0[i, pl.ds(j, 16)] = jnp.zeros((16,), jnp.float32)

        nbase = pl.multiple_of(s * NPS_A, 8)

        def zero_rows(base, nrows):
            hs = []
            for t in range(nrows // WIN):
                hs.append(pltpu.async_copy(
                    r0, acc.at[pl.ds(base + t * WIN, WIN)], giA))
            rem = nrows - (nrows // WIN) * WIN
            if rem:
                hs.append(pltpu.async_copy(
                    r0.at[pl.ds(0, rem)],
                    acc.at[pl.ds(base + (nrows // WIN) * WIN, rem)], giB))
            for h in hs:
                h.wait()

        @pl.when(s < NS - 1)
        def _():
            zero_rows(nbase, NPS_A)

        @pl.when(s == NS - 1)
        def _():
            zero_rows(nbase, NPS_B)

        plsc.subcore_barrier()

        ebase = wid * (WPS * WIN)
        A = ((iaA0, ibA0), (iaA1, ibA1), (iaA2, ibA2))
        B = ((iaB0, ibB0), (iaB1, ibB1), (iaB2, ibB2))
        GW = 3 * WIN

        def fire_idx(goff, bufs, sem):
            for d, (ia, ib) in enumerate(bufs):
                pltpu.async_copy(ra_hbm.at[pl.ds(goff + d * WIN, WIN)], ia, sem)
                pltpu.async_copy(br_hbm.at[pl.ds(goff + d * WIN, WIN)], ib, sem)

        def drain_idx(goff, bufs, sem):
            for d, (ia, ib) in enumerate(bufs):
                pltpu.make_async_copy(
                    ra_hbm.at[pl.ds(goff + d * WIN, WIN)], ia, sem).wait()
                pltpu.make_async_copy(
                    br_hbm.at[pl.ds(goff + d * WIN, WIN)], ib, sem).wait()

        def process(bufs):
            (j0, k0), (j1, k1), (j2, k2) = bufs
            cp0 = pltpu.async_copy(x_hbm.at[j0], r0, g0)
            cp1 = pltpu.async_copy(x_hbm.at[j1], r1, g1)
            cp2 = pltpu.async_copy(x_hbm.at[j2], r2, g2)
            cp0.wait()
            pltpu.sync_copy(r0, acc.at[k0], add=True)
            cp1.wait()
            pltpu.sync_copy(r1, acc.at[k1], add=True)
            cp2.wait()
            pltpu.sync_copy(r2, acc.at[k2], add=True)

        fire_idx(ebase, A, giA)

        @pl.loop(0, NDUO)
        def _(q):
            offA = ebase + q * 2 * GW
            offB = offA + GW
            fire_idx(offB, B, giB)
            drain_idx(offA, A, giA)
            process(A)

            @pl.when(q < NDUO - 1)
            def _():
                fire_idx(offA + 2 * GW, A, giA)

            drain_idx(offB, B, giB)
            process(B)

        @pl.when(wid >= NW - NXTRA)
        def _():
            off = (WPS * NW + (wid - (NW - NXTRA))) * WIN
            pltpu.sync_copy(ra_hbm.at[pl.ds(off, WIN)], iaA0)
            pltpu.sync_copy(br_hbm.at[pl.ds(off, WIN)], ibA0)
            pltpu.async_copy(x_hbm.at[iaA0], r0, g0).wait()
            pltpu.sync_copy(r0, acc.at[ibA0], add=True)

        plsc.subcore_barrier()

        @pl.when(s < NS - 1)
        def _():
            pltpu.sync_copy(acc.at[pl.ds(nbase, NPS_A)],
                            out_hbm.at[c, pl.ds(nbase, NPS_A)])

        @pl.when(s == NS - 1)
        def _():
            pltpu.sync_copy(acc.at[pl.ds(nbase, NPS_B)],
                            out_hbm.at[c, pl.ds(nbase, NPS_B)])

    return k(x, ref_a, backref)


def _tc_combine(x, s0, s1, w, w_prop, b):
    br = 1000

    def body(x_ref, s0_ref, s1_ref, w_ref, wp_ref, b_ref, o_ref):
        acc = jnp.dot(x_ref[...], w_ref[...], preferred_element_type=jnp.float32)
        conv = s0_ref[...] + s1_ref[...]
        acc += jnp.dot(conv, wp_ref[...], preferred_element_type=jnp.float32)
        o_ref[...] = jnp.maximum(acc + b_ref[...], 0.0)

    return pl.pallas_call(
        body,
        grid=(N_NODES // br,),
        in_specs=[
            pl.BlockSpec((br, D), lambda i: (i, 0)),
            pl.BlockSpec((br, D), lambda i: (i, 0)),
            pl.BlockSpec((br, D), lambda i: (i, 0)),
            pl.BlockSpec((D, D), lambda i: (0, 0)),
            pl.BlockSpec((D, D), lambda i: (0, 0)),
            pl.BlockSpec((1, D), lambda i: (0, 0)),
        ],
        out_specs=pl.BlockSpec((br, D), lambda i: (i, 0)),
        out_shape=jax.ShapeDtypeStruct((N_NODES, D), jnp.float32),
    )(x, s0, s1, w, w_prop, b.reshape(1, D))


def kernel(X, ref_a, backref, e_map, v_count, W, W_prop, b):
    partials = _sc_gather_segment_sum(X, ref_a, backref)
    X_out = _tc_combine(X, partials[0], partials[1], W, W_prop, b)
    return (X_out, ref_a, backref, e_map, v_count)
